# Initial kernel scaffold; baseline (speedup 1.0000x reference)
import jax
import jax.numpy as jnp
from jax.experimental import pallas as pl


def kernel(scores, nBestIndex, werRank):
    raise NotImplementedError("write your pallas kernel here")


# speedup vs baseline: 1.0331x; 1.0331x over previous
import jax
import jax.numpy as jnp
from jax import lax
from jax.experimental import pallas as pl
from jax.experimental.pallas import tpu as pltpu
from jax.experimental.pallas import tpu_sc as plsc

B = 16
L = 2048
MARGIN = 0.1
LANES = 16
NSUB = 16
NCHUNK = L // 128


def _sc_body(scores_hbm, wr_hbm, part_hbm, idx_v, vals_v, acc_v, sem):
    wid = lax.axis_index("s")

    pltpu.sync_copy(wr_hbm.at[pl.ds(wid * NCHUNK, NCHUNK)], idx_v)
    copies = [
        pltpu.async_copy(scores_hbm.at[idx_v.at[i]], vals_v.at[i], sem)
        for i in range(NCHUNK)
    ]
    for c in copies:
        c.wait()

    top = vals_v[0, pl.ds(0, LANES)][0]

    acc = jnp.zeros((LANES,), jnp.float32)
    for i in range(NCHUNK):
        for k in range(128 // LANES):
            x = vals_v[i, pl.ds(k * LANES, LANES)]
            acc = acc + jnp.maximum(x - top + MARGIN, 0.0)
    acc_v[...] = acc

    pltpu.sync_copy(acc_v, part_hbm.at[wid])


_mesh = plsc.VectorSubcoreMesh(core_axis_name="c", subcore_axis_name="s",
                               num_cores=1, num_subcores=NSUB)

_sc_call = pl.kernel(
    _sc_body,
    out_type=jax.ShapeDtypeStruct((NSUB, LANES), jnp.float32),
    mesh=_mesh,
    scratch_types=[
        pltpu.VMEM((NCHUNK, 128), jnp.int32),
        pltpu.VMEM((NCHUNK, 128), jnp.float32),
        pltpu.VMEM((LANES,), jnp.float32),
        pltpu.SemaphoreType.DMA,
    ],
    name="self_margin_loss_sc",
)


def _tc_body(part_ref, out_ref):
    s = jnp.sum(part_ref[...])
    out_ref[...] = jnp.full((1, 1), (s - B * MARGIN) * (1.0 / (L - 1)),
                            jnp.float32)


_tc_call = pl.pallas_call(
    _tc_body,
    out_shape=jax.ShapeDtypeStruct((1, 1), jnp.float32),
)


@jax.jit
def kernel(scores, nBestIndex, werRank):
    del nBestIndex
    parts = _sc_call(scores, werRank.reshape(-1, 128))
    return _tc_call(parts).reshape(1)


# speedup vs baseline: 1.0721x; 1.0377x over previous
import jax
import jax.numpy as jnp
from jax import lax
from jax.experimental import pallas as pl
from jax.experimental.pallas import tpu as pltpu
from jax.experimental.pallas import tpu_sc as plsc

B = 16
L = 2048
MARGIN = 0.1
LANES = 16
NSUB = 16
NCHUNK = L // 128


def _sc_body(scores_hbm, wr_hbm, out_hbm, idx_v, vals_v, acc_v, zidx_v,
             shared, red_v, out_v, sem):
    wid = lax.axis_index("s")

    pltpu.sync_copy(wr_hbm.at[pl.ds(wid * NCHUNK, NCHUNK)], idx_v)
    copies = [
        pltpu.async_copy(scores_hbm.at[idx_v.at[i]], vals_v.at[i], sem)
        for i in range(NCHUNK)
    ]

    zidx_v[...] = jnp.zeros((LANES,), jnp.int32)

    @pl.when(wid == 0)
    def _():
        out_v[...] = jnp.zeros((LANES,), jnp.float32)
        pltpu.sync_copy(out_v, shared.at[0])

    for c in copies:
        c.wait()

    top = vals_v[0, pl.ds(0, LANES)][0]

    acc = jnp.zeros((LANES,), jnp.float32)
    for i in range(NCHUNK):
        for k in range(128 // LANES):
            x = vals_v[i, pl.ds(k * LANES, LANES)]
            acc = acc + jnp.maximum(x - top + MARGIN, 0.0)
    acc_v[0, pl.ds(0, LANES)] = acc

    plsc.subcore_barrier()
    pltpu.sync_copy(acc_v, shared.at[zidx_v.at[pl.ds(0, 1)]], add=True)
    plsc.subcore_barrier()

    @pl.when(wid == 0)
    def _():
        pltpu.sync_copy(shared.at[0], red_v)
        tot = red_v[pl.ds(0, LANES)]
        s = tot[0]
        for i in range(1, LANES):
            s = s + tot[i]
        final = (s - B * MARGIN) * (1.0 / (L - 1))
        out_v[...] = jnp.full((LANES,), final, jnp.float32)
        pltpu.sync_copy(out_v, out_hbm)


_mesh = plsc.VectorSubcoreMesh(core_axis_name="c", subcore_axis_name="s",
                               num_cores=1, num_subcores=NSUB)

_sc_call = pl.kernel(
    _sc_body,
    out_type=jax.ShapeDtypeStruct((LANES,), jnp.float32),
    mesh=_mesh,
    scratch_types=[
        pltpu.VMEM((NCHUNK, 128), jnp.int32),
        pltpu.VMEM((NCHUNK, 128), jnp.float32),
        pltpu.VMEM((1, LANES), jnp.float32),
        pltpu.VMEM((LANES,), jnp.int32),
        pltpu.VMEM_SHARED((1, LANES), jnp.float32),
        pltpu.VMEM((LANES,), jnp.float32),
        pltpu.VMEM((LANES,), jnp.float32),
        pltpu.SemaphoreType.DMA,
    ],
    name="self_margin_loss_sc",
)


@jax.jit
def kernel(scores, nBestIndex, werRank):
    del nBestIndex
    out = _sc_call(scores, werRank.reshape(-1, 128))
    return out[0:1]
